# Initial kernel scaffold; baseline (speedup 1.0000x reference)
import jax
import jax.numpy as jnp
from jax.experimental import pallas as pl


def kernel(hidden_states, attention_mask, position_ids, ln_weight, router_w, W_gate, W_up, W_down):
    raise NotImplementedError("write your pallas kernel here")


# speedup vs baseline: 2.0789x; 2.0789x over previous
import functools

import jax
import jax.numpy as jnp
from jax.experimental import pallas as pl
from jax.experimental.pallas import tpu as pltpu

B, S, D = 1, 2048, 768
E, K, F = 64, 2, 128
EPS = 1e-6
T = B * S


def _route_body(x_ref, lnw_ref, rw_ref, h_ref, i1_ref, i2_ref, w1_ref, w2_ref):
    x = x_ref[...]
    v = jnp.mean(x * x, axis=-1, keepdims=True)
    h = x * jax.lax.rsqrt(v + EPS) * lnw_ref[...]
    h_ref[...] = h
    logits = jnp.dot(h, rw_ref[...], preferred_element_type=jnp.float32)
    iota = jax.lax.broadcasted_iota(jnp.int32, logits.shape, 1)
    m1 = jnp.max(logits, axis=-1, keepdims=True)
    i1 = jnp.min(jnp.where(logits == m1, iota, E), axis=-1, keepdims=True)
    l2 = jnp.where(iota == i1, -jnp.inf, logits)
    m2 = jnp.max(l2, axis=-1, keepdims=True)
    i2 = jnp.min(jnp.where(l2 == m2, iota, E), axis=-1, keepdims=True)
    e2 = jnp.exp(m2 - m1)
    s = 1.0 + e2
    i1_ref[...] = i1
    i2_ref[...] = i2
    w1_ref[...] = 1.0 / s
    w2_ref[...] = e2 / s


def _route(x, ln_weight, router_w):
    return pl.pallas_call(
        _route_body,
        out_shape=(
            jax.ShapeDtypeStruct((T, D), jnp.float32),
            jax.ShapeDtypeStruct((T, 1), jnp.int32),
            jax.ShapeDtypeStruct((T, 1), jnp.int32),
            jax.ShapeDtypeStruct((T, 1), jnp.float32),
            jax.ShapeDtypeStruct((T, 1), jnp.float32),
        ),
    )(x, ln_weight.reshape(1, D), router_w)


def _moe_body(h_ref, i1_ref, i2_ref, w1_ref, w2_ref, wg_ref, wu_ref, wd_ref,
              out_ref):
    e = pl.program_id(0)
    h = h_ref[...]
    g = jnp.dot(h, wg_ref[0], preferred_element_type=jnp.float32)
    u = jnp.dot(h, wu_ref[0], preferred_element_type=jnp.float32)
    a = (g * jax.nn.sigmoid(g)) * u
    y = jnp.dot(a, wd_ref[0], preferred_element_type=jnp.float32)
    w_col = (jnp.where(i1_ref[...] == e, w1_ref[...], 0.0)
             + jnp.where(i2_ref[...] == e, w2_ref[...], 0.0))
    contrib = w_col * y

    @pl.when(e == 0)
    def _():
        out_ref[...] = contrib

    @pl.when(e > 0)
    def _():
        out_ref[...] += contrib


def _moe_dense(h, i1, i2, w1, w2, W_gate, W_up, W_down):
    return pl.pallas_call(
        _moe_body,
        grid=(E,),
        in_specs=[
            pl.BlockSpec((T, D), lambda e: (0, 0)),
            pl.BlockSpec((T, 1), lambda e: (0, 0)),
            pl.BlockSpec((T, 1), lambda e: (0, 0)),
            pl.BlockSpec((T, 1), lambda e: (0, 0)),
            pl.BlockSpec((T, 1), lambda e: (0, 0)),
            pl.BlockSpec((1, D, F), lambda e: (e, 0, 0)),
            pl.BlockSpec((1, D, F), lambda e: (e, 0, 0)),
            pl.BlockSpec((1, F, D), lambda e: (e, 0, 0)),
        ],
        out_specs=pl.BlockSpec((T, D), lambda e: (0, 0)),
        out_shape=jax.ShapeDtypeStruct((T, D), jnp.float32),
    )(h, i1, i2, w1, w2, W_gate, W_up, W_down)


def kernel(hidden_states, attention_mask, position_ids, ln_weight, router_w,
           W_gate, W_up, W_down):
    x = hidden_states.reshape(T, D)
    h, i1, i2, w1, w2 = _route(x, ln_weight, router_w)
    moe = _moe_dense(h, i1, i2, w1, w2, W_gate, W_up, W_down)
    return hidden_states + moe.reshape(B, S, D)
